# baseline (speedup 1.0000x reference)
import jax
import jax.numpy as jnp
from jax import lax
from jax.experimental import pallas as pl
from jax.experimental.pallas import tpu as pltpu, tpu_sc as plsc

D = 1536
N_BINS = 16
_SEQ_V, _SS8_V, _SASA_V, _FUNC_V = 64, 11, 19, 260
_STRUCT_V, _RES_V = 4101, 1478


_TT = 1
_RPT = 17
_IPC = 24
_NW = 32


def _sc_body(table, idx_hbm, out, idx_v, rows0, rows1, st0, st1,
             gsem0, gsem1, osem0, osem1):
    c = lax.axis_index("c")
    s = lax.axis_index("s")
    wid = s * 2 + c
    n_chunks = 8192 // _NW // _TT
    rows = [rows0, rows1]
    stage = [st0, st1]
    gsems = [gsem0, gsem1]
    osems = [osem0, osem1]

    pltpu.sync_copy(idx_hbm.at[pl.ds(wid * n_chunks * _IPC, n_chunks * _IPC)],
                    idx_v)

    def issue_gather(ci, bi):
        pltpu.async_copy(
            table.at[idx_v.at[pl.ds(ci * _IPC, _IPC)]], rows[bi],
            gsems[bi],
        )

    def wait_gather(bi):
        pltpu.make_async_copy(
            table.at[idx_v.at[pl.ds(0, _IPC)]], rows[bi], gsems[bi]
        ).wait()

    def issue_out(ci, bi):
        base = (wid * n_chunks + ci) * _TT * D
        pltpu.async_copy(stage[bi], out.at[pl.ds(base, _TT * D)], osems[bi])

    def wait_out(bi):
        pltpu.make_async_copy(
            stage[bi], out.at[pl.ds(0, _TT * D)], osems[bi]
        ).wait()

    def compute(bi):
        buf = rows[bi]
        stg = stage[bi]

        def piece(j, c2):
            sl = pl.ds(pl.multiple_of(j * 16, 16), 16)
            for t in range(_TT):
                v = buf[t * _RPT, sl]
                for k in range(1, _RPT):
                    v = v + buf[t * _RPT + k, sl]
                stg[pl.ds(pl.multiple_of(t * D + j * 16, 16), 16)] = v
            return c2

        lax.fori_loop(0, D // 16, piece, 0)

    issue_gather(0, 0)
    issue_gather(1, 1)

    for bi in (0, 1):
        wait_gather(bi)
        compute(bi)
        issue_out(bi, bi)
        issue_gather(bi + 2, bi)

    def pair(p, carry):
        for bi in (0, 1):
            ci = p * 2 + bi
            wait_gather(bi)
            wait_out(bi)
            compute(bi)
            issue_out(ci, bi)

            @pl.when(ci + 2 < n_chunks)
            def _():
                issue_gather(ci + 2, bi)

        return carry

    lax.fori_loop(1, n_chunks // 2, pair, 0)
    wait_out(0)
    wait_out(1)


def _sc_gather(table, idx_flat, n):
    mesh = plsc.VectorSubcoreMesh(
        core_axis_name="c", subcore_axis_name="s", num_cores=2, num_subcores=16
    )
    fn = pl.kernel(
        _sc_body,
        out_type=jax.ShapeDtypeStruct((n * D,), jnp.float32),
        mesh=mesh,
        scratch_types=[
            pltpu.VMEM((n // _NW // _TT * _IPC,), jnp.int32),
            pltpu.VMEM((_IPC, D), jnp.float32),
            pltpu.VMEM((_IPC, D), jnp.float32),
            pltpu.VMEM((_TT * D,), jnp.float32),
            pltpu.VMEM((_TT * D,), jnp.float32),
            pltpu.SemaphoreType.DMA,
            pltpu.SemaphoreType.DMA,
            pltpu.SemaphoreType.DMA,
            pltpu.SemaphoreType.DMA,
        ],
    )
    return fn(table, idx_flat)


_BT = 512


def _epi_body(part_ref, avg_ref, per_ref, seq_ref, ss8_ref, sasa_ref,
              func_ref, wsmall_ref, ftab_ref, bias_ref, o_ref):
    j = lax.broadcasted_iota(jnp.int32, (_BT, 128), 1)
    jf = j.astype(jnp.float32)
    xa = jnp.broadcast_to(avg_ref[:], (_BT, 128))
    xp = jnp.broadcast_to(per_ref[:], (_BT, 128))
    za = (xa - (jf - 94.0) * (1.0 / (N_BINS - 1))) * float(N_BINS)
    zp = (xp - (jf - 110.0) * (1.0 / (N_BINS - 1))) * float(N_BINS)
    f_seq = (j == jnp.broadcast_to(seq_ref[:], (_BT, 128))).astype(jnp.float32)
    f_ss8 = (j - 64 == jnp.broadcast_to(ss8_ref[:], (_BT, 128))).astype(jnp.float32)
    f_sasa = (j - 75 == jnp.broadcast_to(sasa_ref[:], (_BT, 128))).astype(jnp.float32)
    f = jnp.where(
        j < 64, f_seq,
        jnp.where(
            j < 75, f_ss8,
            jnp.where(
                j < 94, f_sasa,
                jnp.where(j < 110, jnp.exp(-za * za),
                          jnp.where(j < 126, jnp.exp(-zp * zp), 0.0)),
            ),
        ),
    )
    acc = lax.dot_general(
        f, wsmall_ref[:], (((1,), (0,)), ((), ())),
        preferred_element_type=jnp.float32, precision=lax.Precision.DEFAULT,
    )
    pieces = []
    for i in range(8):
        oh = (
            lax.broadcasted_iota(jnp.int32, (_BT, _FUNC_V), 1)
            == jnp.broadcast_to(func_ref[:, i : i + 1], (_BT, _FUNC_V))
        ).astype(jnp.float32)
        pieces.append(
            lax.dot_general(
                oh, ftab_ref[i * _FUNC_V : (i + 1) * _FUNC_V, :],
                (((1,), (0,)), ((), ())),
                preferred_element_type=jnp.float32,
                precision=lax.Precision.DEFAULT,
            )
        )
    o_ref[:] = part_ref[:] + acc + jnp.concatenate(pieces, axis=1) + bias_ref[:]


def _epilogue(part, avg, per, seq, ss8, sasa, func_tok, wsmall, ftab, bias, n):
    col = lambda i: (i, 0)
    full = lambda i: (0, 0)
    return pl.pallas_call(
        _epi_body,
        grid=(n // _BT,),
        in_specs=[
            pl.BlockSpec((_BT, D), col),
            pl.BlockSpec((_BT, 1), col),
            pl.BlockSpec((_BT, 1), col),
            pl.BlockSpec((_BT, 1), col),
            pl.BlockSpec((_BT, 1), col),
            pl.BlockSpec((_BT, 1), col),
            pl.BlockSpec((_BT, 8), col),
            pl.BlockSpec((128, D), full),
            pl.BlockSpec((8 * _FUNC_V, D // 8), full),
            pl.BlockSpec((1, D), full),
        ],
        out_specs=pl.BlockSpec((_BT, D), col),
        out_shape=jax.ShapeDtypeStruct((n, D), jnp.float32),
    )(part, avg, per, seq, ss8, sasa, func_tok, wsmall, ftab, bias)


def kernel(sequence_tokens, structure_tokens, average_plddt, per_res_plddt,
           ss8_tokens, sasa_tokens, function_tokens, residue_annotation_tokens,
           seq_table, struct_table, ss8_table, sasa_table, func_tables,
           residue_table, plddt_W, plddt_b, perres_W, perres_b):
    B, L = sequence_tokens.shape
    n = B * L

    table = jnp.concatenate([struct_table, residue_table], axis=0)
    idx17 = jnp.concatenate(
        [
            structure_tokens.reshape(n, 1),
            residue_annotation_tokens.reshape(n, 16) + _STRUCT_V,
        ],
        axis=1,
    ).astype(jnp.int32)
    idx_flat = jnp.pad(
        idx17.reshape(n // _TT, _RPT * _TT), ((0, 0), (0, _IPC - _RPT * _TT))
    ).reshape(n // _TT * _IPC)
    part = _sc_gather(table, idx_flat, n).reshape(n, D)

    wsmall = jnp.concatenate(
        [seq_table, ss8_table, sasa_table, plddt_W, perres_W,
         jnp.zeros((2, D), jnp.float32)],
        axis=0,
    )
    bias = (plddt_b + perres_b).reshape(1, D)
    out = _epilogue(
        part,
        average_plddt.reshape(n, 1),
        per_res_plddt.reshape(n, 1),
        sequence_tokens.reshape(n, 1).astype(jnp.int32),
        ss8_tokens.reshape(n, 1).astype(jnp.int32),
        sasa_tokens.reshape(n, 1).astype(jnp.int32),
        function_tokens.reshape(n, 8).astype(jnp.int32),
        wsmall,
        func_tables.reshape(8 * _FUNC_V, D // 8),
        bias,
        n,
    )
    return out.reshape(B, L, D)
